# Initial kernel scaffold; baseline (speedup 1.0000x reference)
import jax
import jax.numpy as jnp
from jax.experimental import pallas as pl


def kernel(atom_fea, nbr_fea, nbr_idx, W, b, gamma, beta):
    raise NotImplementedError("write your pallas kernel here")


# speedup vs baseline: 2.8432x; 2.8432x over previous
import functools

import jax
import jax.numpy as jnp
from jax import lax
from jax.experimental import pallas as pl
from jax.experimental.pallas import tpu as pltpu
from jax.experimental.pallas import tpu_sc as plsc

N, M, A, E = 10000, 32, 128, 16
NM = N * M

_NC, _NS = 2, 16
_NW = _NC * _NS
_BPW = NM // _NW
_BC = 80
_CHUNKS = _BPW // _BC

@functools.lru_cache(maxsize=1)
def _sc_gather():
    mesh = plsc.VectorSubcoreMesh(core_axis_name="c", subcore_axis_name="s")

    @functools.partial(
        pl.kernel,
        out_type=jax.ShapeDtypeStruct((NM, A), jnp.float32),
        mesh=mesh,
        scratch_types=[
            pltpu.VMEM((_BC,), jnp.int32),
            pltpu.VMEM((_BC, A), jnp.float32),
            pltpu.SemaphoreType.DMA,
        ],
    )
    def gather(table_hbm, idx_hbm, out_hbm, idx_v, rows_v, sem):
        wid = lax.axis_index("s") * _NC + lax.axis_index("c")
        base = wid * _BPW

        def body(c, carry):
            off = base + c * _BC
            pltpu.sync_copy(idx_hbm.at[pl.ds(off, _BC)], idx_v)
            pltpu.async_copy(table_hbm.at[idx_v], rows_v, sem).wait()
            pltpu.sync_copy(rows_v, out_hbm.at[pl.ds(off, _BC)])
            return carry

        lax.fori_loop(0, _CHUNKS, body, 0)

    return gather


_BLK = 400


def _softplus(x):
    return jnp.maximum(x, 0.0) + jnp.log(1.0 + jnp.exp(-jnp.abs(x)))


def _pass1_body(g_ref, a_ref, nbr_ref, ws_ref, wn_ref, we_ref, b_ref,
                pre_ref, ssum_ref, ssq_ref):
    i = pl.program_id(0)
    atom = a_ref[...]
    s = jnp.dot(atom, ws_ref[...], preferred_element_type=jnp.float32)
    s = s + b_ref[0:1, :]
    wn = wn_ref[...]
    we = we_ref[...]
    acc = jnp.zeros((_BLK, A), jnp.float32)
    for j in range(M):
        z = s + jnp.dot(g_ref[j], wn, preferred_element_type=jnp.float32)
        z = z + jnp.dot(nbr_ref[:, j * E:(j + 1) * E], we,
                        preferred_element_type=jnp.float32)
        f = z[:, :A]
        c = z[:, A:]
        sig = 1.0 / (1.0 + jnp.exp(-f))
        acc = acc + sig * _softplus(c)
    pre = atom + acc
    pre_ref[...] = pre

    @pl.when(i == 0)
    def _():
        ssum_ref[...] = jnp.zeros_like(ssum_ref)
        ssq_ref[...] = jnp.zeros_like(ssq_ref)

    ssum_ref[0:1, :] += jnp.sum(pre, axis=0, keepdims=True)
    ssq_ref[0:1, :] += jnp.sum(pre * pre, axis=0, keepdims=True)


_BLK2 = 1000


def _pass2_body(pre_ref, ssum_ref, ssq_ref, g_ref, bt_ref, out_ref):
    inv_n = 1.0 / N
    mean = ssum_ref[0:1, :] * inv_n
    var = ssq_ref[0:1, :] * inv_n - mean * mean
    rstd = lax.rsqrt(var + 1e-5)
    x = (pre_ref[...] - mean) * (rstd * g_ref[0:1, :]) + bt_ref[0:1, :]
    out_ref[...] = _softplus(x)


def kernel(atom_fea, nbr_fea, nbr_idx, W, b, gamma, beta):
    idx_flat = nbr_idx.T.reshape(NM)
    gathered = _sc_gather()(atom_fea, idx_flat)
    gathered3 = gathered.reshape(M, N, A)

    nbr_flat = nbr_fea.reshape(N, M * E)
    ws = W[:A]
    wn = W[A:2 * A]
    we = W[2 * A:]
    b8 = jnp.broadcast_to(b.reshape(1, 2 * A), (8, 2 * A))

    pre, ssum, ssq = pl.pallas_call(
        _pass1_body,
        grid=(N // _BLK,),
        in_specs=[
            pl.BlockSpec((M, _BLK, A), lambda i: (0, i, 0)),
            pl.BlockSpec((_BLK, A), lambda i: (i, 0)),
            pl.BlockSpec((_BLK, M * E), lambda i: (i, 0)),
            pl.BlockSpec((A, 2 * A), lambda i: (0, 0)),
            pl.BlockSpec((A, 2 * A), lambda i: (0, 0)),
            pl.BlockSpec((E, 2 * A), lambda i: (0, 0)),
            pl.BlockSpec((8, 2 * A), lambda i: (0, 0)),
        ],
        out_specs=[
            pl.BlockSpec((_BLK, A), lambda i: (i, 0)),
            pl.BlockSpec((8, A), lambda i: (0, 0)),
            pl.BlockSpec((8, A), lambda i: (0, 0)),
        ],
        out_shape=[
            jax.ShapeDtypeStruct((N, A), jnp.float32),
            jax.ShapeDtypeStruct((8, A), jnp.float32),
            jax.ShapeDtypeStruct((8, A), jnp.float32),
        ],
    )(gathered3, atom_fea, nbr_flat, ws, wn, we, b8)

    g8 = jnp.broadcast_to(gamma.reshape(1, A), (8, A))
    bt8 = jnp.broadcast_to(beta.reshape(1, A), (8, A))
    out = pl.pallas_call(
        _pass2_body,
        grid=(N // _BLK2,),
        in_specs=[
            pl.BlockSpec((_BLK2, A), lambda i: (i, 0)),
            pl.BlockSpec((8, A), lambda i: (0, 0)),
            pl.BlockSpec((8, A), lambda i: (0, 0)),
            pl.BlockSpec((8, A), lambda i: (0, 0)),
            pl.BlockSpec((8, A), lambda i: (0, 0)),
        ],
        out_specs=pl.BlockSpec((_BLK2, A), lambda i: (i, 0)),
        out_shape=jax.ShapeDtypeStruct((N, A), jnp.float32),
    )(pre, ssum, ssq, g8, bt8)
    return out


# speedup vs baseline: 4.2484x; 1.4942x over previous
import functools

import jax
import jax.numpy as jnp
from jax import lax
from jax.experimental import pallas as pl
from jax.experimental.pallas import tpu as pltpu
from jax.experimental.pallas import tpu_sc as plsc

N, M, A, E = 10000, 32, 128, 16
NM = N * M

_NC, _NS = 2, 16
_NW = _NC * _NS
_BPW = NM // _NW
_BC = 80
_CHUNKS = _BPW // _BC

_NB = 5


@functools.lru_cache(maxsize=1)
def _sc_gather():
    mesh = plsc.VectorSubcoreMesh(core_axis_name="c", subcore_axis_name="s")
    ngroups = _CHUNKS // _NB

    @functools.partial(
        pl.kernel,
        out_type=jax.ShapeDtypeStruct((NM, A), jnp.float32),
        mesh=mesh,
        scratch_types=[
            pltpu.VMEM((_BPW,), jnp.int32),
            pltpu.VMEM((_NB, _BC, A), jnp.float32),
        ] + [pltpu.SemaphoreType.DMA] * (2 * _NB),
    )
    def gather(table_hbm, idx_hbm, out_hbm, idx_v, bufs, *sems):
        gsem, wsem = sems[:_NB], sems[_NB:]
        wid = lax.axis_index("s") * _NC + lax.axis_index("c")
        base = wid * _BPW
        pltpu.sync_copy(idx_hbm.at[pl.ds(base, _BPW)], idx_v)

        def issue_gather(off, b):
            off = pl.multiple_of(off, 8)
            pltpu.async_copy(table_hbm.at[idx_v.at[pl.ds(off, _BC)]],
                             bufs.at[b], gsem[b])

        def wait_gather(b):
            pltpu.make_async_copy(table_hbm.at[pl.ds(0, _BC)],
                                  bufs.at[b], gsem[b]).wait()

        def issue_write(off, b):
            pltpu.async_copy(bufs.at[b], out_hbm.at[pl.ds(base + off, _BC)],
                             wsem[b])

        def wait_write(b):
            pltpu.make_async_copy(bufs.at[b], out_hbm.at[pl.ds(base, _BC)],
                                  wsem[b]).wait()

        for b in range(_NB):
            issue_gather(b * _BC, b)

        def group(g, carry):
            for b in range(_NB):
                wait_gather(b)
                issue_write((g * _NB + b) * _BC, b)
            for b in range(_NB):
                wait_write(b)
                issue_gather(((g + 1) * _NB + b) * _BC, b)
            return carry

        lax.fori_loop(0, ngroups - 1, group, 0)
        g_last = ngroups - 1
        for b in range(_NB):
            wait_gather(b)
            issue_write((g_last * _NB + b) * _BC, b)
        for b in range(_NB):
            wait_write(b)

    return gather


_BLK = 400


def _softplus(x):
    return jnp.maximum(x, 0.0) + jnp.log(1.0 + jnp.exp(-jnp.abs(x)))


def _pass1_body(g_ref, a_ref, nbr_ref, ws_ref, wn_ref, we_ref, b_ref,
                pre_ref, ssum_ref, ssq_ref):
    i = pl.program_id(0)
    atom = a_ref[...]
    s = jnp.dot(atom, ws_ref[...], preferred_element_type=jnp.float32)
    s = s + b_ref[0:1, :]
    wn = wn_ref[...]
    we = we_ref[...]
    acc = jnp.zeros((_BLK, A), jnp.float32)
    for j in range(M):
        z = s + jnp.dot(g_ref[j], wn, preferred_element_type=jnp.float32)
        z = z + jnp.dot(nbr_ref[:, j * E:(j + 1) * E], we,
                        preferred_element_type=jnp.float32)
        f = z[:, :A]
        c = z[:, A:]
        sig = 1.0 / (1.0 + jnp.exp(-f))
        acc = acc + sig * _softplus(c)
    pre = atom + acc
    pre_ref[...] = pre

    @pl.when(i == 0)
    def _():
        ssum_ref[...] = jnp.zeros_like(ssum_ref)
        ssq_ref[...] = jnp.zeros_like(ssq_ref)

    ssum_ref[0:1, :] += jnp.sum(pre, axis=0, keepdims=True)
    ssq_ref[0:1, :] += jnp.sum(pre * pre, axis=0, keepdims=True)


_BLK2 = 1000


def _pass2_body(pre_ref, ssum_ref, ssq_ref, g_ref, bt_ref, out_ref):
    inv_n = 1.0 / N
    mean = ssum_ref[0:1, :] * inv_n
    var = ssq_ref[0:1, :] * inv_n - mean * mean
    rstd = lax.rsqrt(var + 1e-5)
    x = (pre_ref[...] - mean) * (rstd * g_ref[0:1, :]) + bt_ref[0:1, :]
    out_ref[...] = _softplus(x)


def kernel(atom_fea, nbr_fea, nbr_idx, W, b, gamma, beta):
    idx_flat = nbr_idx.T.reshape(NM)
    gathered = _sc_gather()(atom_fea, idx_flat)
    gathered3 = gathered.reshape(M, N, A)

    nbr_flat = nbr_fea.reshape(N, M * E)
    ws = W[:A]
    wn = W[A:2 * A]
    we = W[2 * A:]
    b8 = jnp.broadcast_to(b.reshape(1, 2 * A), (8, 2 * A))

    pre, ssum, ssq = pl.pallas_call(
        _pass1_body,
        grid=(N // _BLK,),
        in_specs=[
            pl.BlockSpec((M, _BLK, A), lambda i: (0, i, 0)),
            pl.BlockSpec((_BLK, A), lambda i: (i, 0)),
            pl.BlockSpec((_BLK, M * E), lambda i: (i, 0)),
            pl.BlockSpec((A, 2 * A), lambda i: (0, 0)),
            pl.BlockSpec((A, 2 * A), lambda i: (0, 0)),
            pl.BlockSpec((E, 2 * A), lambda i: (0, 0)),
            pl.BlockSpec((8, 2 * A), lambda i: (0, 0)),
        ],
        out_specs=[
            pl.BlockSpec((_BLK, A), lambda i: (i, 0)),
            pl.BlockSpec((8, A), lambda i: (0, 0)),
            pl.BlockSpec((8, A), lambda i: (0, 0)),
        ],
        out_shape=[
            jax.ShapeDtypeStruct((N, A), jnp.float32),
            jax.ShapeDtypeStruct((8, A), jnp.float32),
            jax.ShapeDtypeStruct((8, A), jnp.float32),
        ],
    )(gathered3, atom_fea, nbr_flat, ws, wn, we, b8)

    g8 = jnp.broadcast_to(gamma.reshape(1, A), (8, A))
    bt8 = jnp.broadcast_to(beta.reshape(1, A), (8, A))
    out = pl.pallas_call(
        _pass2_body,
        grid=(N // _BLK2,),
        in_specs=[
            pl.BlockSpec((_BLK2, A), lambda i: (i, 0)),
            pl.BlockSpec((8, A), lambda i: (0, 0)),
            pl.BlockSpec((8, A), lambda i: (0, 0)),
            pl.BlockSpec((8, A), lambda i: (0, 0)),
            pl.BlockSpec((8, A), lambda i: (0, 0)),
        ],
        out_specs=pl.BlockSpec((_BLK2, A), lambda i: (i, 0)),
        out_shape=jax.ShapeDtypeStruct((N, A), jnp.float32),
    )(pre, ssum, ssq, g8, bt8)
    return out


# speedup vs baseline: 4.3322x; 1.0197x over previous
import functools

import jax
import jax.numpy as jnp
from jax import lax
from jax.experimental import pallas as pl
from jax.experimental.pallas import tpu as pltpu
from jax.experimental.pallas import tpu_sc as plsc

N, M, A, E = 10000, 32, 128, 16
NM = N * M

_NC, _NS = 2, 16
_NW = _NC * _NS
_BPW = NM // _NW
_BC = 80
_CHUNKS = _BPW // _BC

_NB = 5


@functools.lru_cache(maxsize=1)
def _sc_gather():
    mesh = plsc.VectorSubcoreMesh(core_axis_name="c", subcore_axis_name="s")
    ngroups = _CHUNKS // _NB

    @functools.partial(
        pl.kernel,
        out_type=jax.ShapeDtypeStruct((NM, A), jnp.float32),
        mesh=mesh,
        scratch_types=[
            pltpu.VMEM((_BPW,), jnp.int32),
            pltpu.VMEM((_NB, _BC, A), jnp.float32),
        ] + [pltpu.SemaphoreType.DMA] * (2 * _NB),
    )
    def gather(table_hbm, idx_hbm, out_hbm, idx_v, bufs, *sems):
        gsem, wsem = sems[:_NB], sems[_NB:]
        wid = lax.axis_index("s") * _NC + lax.axis_index("c")
        base = wid * _BPW
        pltpu.sync_copy(idx_hbm.at[pl.ds(base, _BPW)], idx_v)

        def issue_gather(off, b):
            off = pl.multiple_of(off, 8)
            pltpu.async_copy(table_hbm.at[idx_v.at[pl.ds(off, _BC)]],
                             bufs.at[b], gsem[b])

        def wait_gather(b):
            pltpu.make_async_copy(table_hbm.at[pl.ds(0, _BC)],
                                  bufs.at[b], gsem[b]).wait()

        def issue_write(off, b):
            pltpu.async_copy(bufs.at[b], out_hbm.at[pl.ds(base + off, _BC)],
                             wsem[b])

        def wait_write(b):
            pltpu.make_async_copy(bufs.at[b], out_hbm.at[pl.ds(base, _BC)],
                                  wsem[b]).wait()

        for b in range(_NB):
            issue_gather(b * _BC, b)

        def group(g, carry):
            for b in range(_NB):
                wait_gather(b)
                issue_write((g * _NB + b) * _BC, b)
            for b in range(_NB):
                wait_write(b)
                issue_gather(((g + 1) * _NB + b) * _BC, b)
            return carry

        lax.fori_loop(0, ngroups - 1, group, 0)
        g_last = ngroups - 1
        for b in range(_NB):
            wait_gather(b)
            issue_write((g_last * _NB + b) * _BC, b)
        for b in range(_NB):
            wait_write(b)

    return gather


_BLK = 400


def _softplus(x):
    return jnp.maximum(x, 0.0) + jnp.log(1.0 + jnp.exp(-jnp.abs(x)))


def _pass1_body(g_ref, a_ref, nbr_ref, ws_ref, wn_ref, we_ref, b_ref,
                pre_ref, ssum_ref, ssq_ref):
    i = pl.program_id(0)
    prec = lax.Precision.DEFAULT
    atom = a_ref[...]
    s = jnp.dot(atom, ws_ref[...], preferred_element_type=jnp.float32,
                precision=prec)
    s = s + b_ref[0:1, :]
    wn = wn_ref[...]
    we = we_ref[...]
    acc = jnp.zeros((_BLK, A), jnp.float32)
    for j in range(M):
        z = s + jnp.dot(g_ref[j], wn, preferred_element_type=jnp.float32,
                        precision=prec)
        z = z + jnp.dot(nbr_ref[:, j * E:(j + 1) * E], we,
                        preferred_element_type=jnp.float32, precision=prec)
        f = z[:, :A]
        c = z[:, A:]
        sig = 0.5 * jnp.tanh(0.5 * f) + 0.5
        acc = acc + sig * _softplus(c)
    pre = atom + acc
    pre_ref[...] = pre

    @pl.when(i == 0)
    def _():
        ssum_ref[...] = jnp.zeros_like(ssum_ref)
        ssq_ref[...] = jnp.zeros_like(ssq_ref)

    ssum_ref[0:1, :] += jnp.sum(pre, axis=0, keepdims=True)
    ssq_ref[0:1, :] += jnp.sum(pre * pre, axis=0, keepdims=True)


_BLK2 = 1000


def _pass2_body(pre_ref, ssum_ref, ssq_ref, g_ref, bt_ref, out_ref):
    inv_n = 1.0 / N
    mean = ssum_ref[0:1, :] * inv_n
    var = ssq_ref[0:1, :] * inv_n - mean * mean
    rstd = lax.rsqrt(var + 1e-5)
    x = (pre_ref[...] - mean) * (rstd * g_ref[0:1, :]) + bt_ref[0:1, :]
    out_ref[...] = _softplus(x)


def kernel(atom_fea, nbr_fea, nbr_idx, W, b, gamma, beta):
    idx_flat = nbr_idx.T.reshape(NM)
    gathered = _sc_gather()(atom_fea, idx_flat)
    gathered3 = gathered.reshape(M, N, A)

    nbr_flat = nbr_fea.reshape(N, M * E)
    ws = W[:A]
    wn = W[A:2 * A]
    we = W[2 * A:]
    b8 = jnp.broadcast_to(b.reshape(1, 2 * A), (8, 2 * A))

    pre, ssum, ssq = pl.pallas_call(
        _pass1_body,
        grid=(N // _BLK,),
        in_specs=[
            pl.BlockSpec((M, _BLK, A), lambda i: (0, i, 0)),
            pl.BlockSpec((_BLK, A), lambda i: (i, 0)),
            pl.BlockSpec((_BLK, M * E), lambda i: (i, 0)),
            pl.BlockSpec((A, 2 * A), lambda i: (0, 0)),
            pl.BlockSpec((A, 2 * A), lambda i: (0, 0)),
            pl.BlockSpec((E, 2 * A), lambda i: (0, 0)),
            pl.BlockSpec((8, 2 * A), lambda i: (0, 0)),
        ],
        out_specs=[
            pl.BlockSpec((_BLK, A), lambda i: (i, 0)),
            pl.BlockSpec((8, A), lambda i: (0, 0)),
            pl.BlockSpec((8, A), lambda i: (0, 0)),
        ],
        out_shape=[
            jax.ShapeDtypeStruct((N, A), jnp.float32),
            jax.ShapeDtypeStruct((8, A), jnp.float32),
            jax.ShapeDtypeStruct((8, A), jnp.float32),
        ],
    )(gathered3, atom_fea, nbr_flat, ws, wn, we, b8)

    g8 = jnp.broadcast_to(gamma.reshape(1, A), (8, A))
    bt8 = jnp.broadcast_to(beta.reshape(1, A), (8, A))
    out = pl.pallas_call(
        _pass2_body,
        grid=(N // _BLK2,),
        in_specs=[
            pl.BlockSpec((_BLK2, A), lambda i: (i, 0)),
            pl.BlockSpec((8, A), lambda i: (0, 0)),
            pl.BlockSpec((8, A), lambda i: (0, 0)),
            pl.BlockSpec((8, A), lambda i: (0, 0)),
            pl.BlockSpec((8, A), lambda i: (0, 0)),
        ],
        out_specs=pl.BlockSpec((_BLK2, A), lambda i: (i, 0)),
        out_shape=jax.ShapeDtypeStruct((N, A), jnp.float32),
    )(pre, ssum, ssq, g8, bt8)
    return out


# speedup vs baseline: 4.7324x; 1.0924x over previous
import functools

import jax
import jax.numpy as jnp
from jax import lax
from jax.experimental import pallas as pl
from jax.experimental.pallas import tpu as pltpu
from jax.experimental.pallas import tpu_sc as plsc

N, M, A, E = 10000, 32, 128, 16
NM = N * M

_NC, _NS = 2, 16
_NW = _NC * _NS
_BPW = NM // _NW
_BC = 80
_CHUNKS = _BPW // _BC

_NB = 5


@functools.lru_cache(maxsize=1)
def _sc_gather():
    mesh = plsc.VectorSubcoreMesh(core_axis_name="c", subcore_axis_name="s")
    ngroups = _CHUNKS // _NB

    @functools.partial(
        pl.kernel,
        out_type=jax.ShapeDtypeStruct((NM, A), jnp.float32),
        mesh=mesh,
        scratch_types=[
            pltpu.VMEM((_BPW,), jnp.int32),
            pltpu.VMEM((_NB, _BC, A), jnp.float32),
        ] + [pltpu.SemaphoreType.DMA] * (2 * _NB),
    )
    def gather(table_hbm, idx_hbm, out_hbm, idx_v, bufs, *sems):
        gsem, wsem = sems[:_NB], sems[_NB:]
        wid = lax.axis_index("s") * _NC + lax.axis_index("c")
        base = wid * _BPW
        pltpu.sync_copy(idx_hbm.at[pl.ds(base, _BPW)], idx_v)

        def issue_gather(off, b):
            off = pl.multiple_of(off, 8)
            pltpu.async_copy(table_hbm.at[idx_v.at[pl.ds(off, _BC)]],
                             bufs.at[b], gsem[b])

        def wait_gather(b):
            pltpu.make_async_copy(table_hbm.at[pl.ds(0, _BC)],
                                  bufs.at[b], gsem[b]).wait()

        def issue_write(off, b):
            pltpu.async_copy(bufs.at[b], out_hbm.at[pl.ds(base + off, _BC)],
                             wsem[b])

        def wait_write(b):
            pltpu.make_async_copy(bufs.at[b], out_hbm.at[pl.ds(base, _BC)],
                                  wsem[b]).wait()

        for b in range(_NB):
            issue_gather(b * _BC, b)

        def group(g, carry):
            for b in range(_NB):
                wait_gather(b)
                issue_write((g * _NB + b) * _BC, b)
            for b in range(_NB):
                wait_write(b)
                issue_gather(((g + 1) * _NB + b) * _BC, b)
            return carry

        lax.fori_loop(0, ngroups - 1, group, 0)
        g_last = ngroups - 1
        for b in range(_NB):
            wait_gather(b)
            issue_write((g_last * _NB + b) * _BC, b)
        for b in range(_NB):
            wait_write(b)

    return gather


_BLK = 400
_LN2 = 0.6931471805599453


def _softplus(x):
    return jnp.maximum(x, 0.0) + jnp.log(1.0 + jnp.exp(-jnp.abs(x)))


def _pass1_body(g_ref, a_ref, nbr_ref, ws_ref, wne_ref, b_ref,
                pre_ref, ssum_ref, ssq_ref):
    i = pl.program_id(0)
    atom = a_ref[...]
    s = jnp.dot(atom.astype(jnp.bfloat16), ws_ref[...],
                preferred_element_type=jnp.float32)
    s = s + b_ref[0:1, :]
    wne = wne_ref[...]
    nbr_bf = nbr_ref[...].astype(jnp.bfloat16)
    acc = jnp.zeros((_BLK, A), jnp.float32)
    for j in range(M):
        x = jnp.concatenate(
            [g_ref[j].astype(jnp.bfloat16), nbr_bf[:, j * E:(j + 1) * E]],
            axis=1)
        z = s + jnp.dot(x, wne, preferred_element_type=jnp.float32)
        fp = z[:, :A]
        cp = z[:, A:]
        t = jnp.maximum(cp, 0.0) + jnp.log2(1.0 + jnp.exp2(-jnp.abs(cp)))
        acc = acc + (1.0 + jnp.tanh(fp)) * t
    pre = atom + (0.5 * _LN2) * acc
    pre_ref[...] = pre

    @pl.when(i == 0)
    def _():
        ssum_ref[...] = jnp.zeros_like(ssum_ref)
        ssq_ref[...] = jnp.zeros_like(ssq_ref)

    ssum_ref[0:1, :] += jnp.sum(pre, axis=0, keepdims=True)
    ssq_ref[0:1, :] += jnp.sum(pre * pre, axis=0, keepdims=True)


_BLK2 = 1000


def _pass2_body(pre_ref, ssum_ref, ssq_ref, g_ref, bt_ref, out_ref):
    inv_n = 1.0 / N
    mean = ssum_ref[0:1, :] * inv_n
    var = ssq_ref[0:1, :] * inv_n - mean * mean
    rstd = lax.rsqrt(var + 1e-5)
    x = (pre_ref[...] - mean) * (rstd * g_ref[0:1, :]) + bt_ref[0:1, :]
    out_ref[...] = _softplus(x)


def kernel(atom_fea, nbr_fea, nbr_idx, W, b, gamma, beta):
    idx_flat = nbr_idx.T.reshape(NM)
    gathered = _sc_gather()(atom_fea, idx_flat)
    gathered3 = gathered.reshape(M, N, A)

    nbr_flat = nbr_fea.reshape(N, M * E)
    colscale = jnp.concatenate(
        [jnp.full((A,), 0.5, jnp.float32),
         jnp.full((A,), 1.4426950408889634, jnp.float32)])
    Ws = W * colscale[None, :]
    ws = Ws[:A].astype(jnp.bfloat16)
    wne = Ws[A:].astype(jnp.bfloat16)
    b8 = jnp.broadcast_to((b * colscale).reshape(1, 2 * A), (8, 2 * A))

    pre, ssum, ssq = pl.pallas_call(
        _pass1_body,
        grid=(N // _BLK,),
        in_specs=[
            pl.BlockSpec((M, _BLK, A), lambda i: (0, i, 0)),
            pl.BlockSpec((_BLK, A), lambda i: (i, 0)),
            pl.BlockSpec((_BLK, M * E), lambda i: (i, 0)),
            pl.BlockSpec((A, 2 * A), lambda i: (0, 0)),
            pl.BlockSpec((A + E, 2 * A), lambda i: (0, 0)),
            pl.BlockSpec((8, 2 * A), lambda i: (0, 0)),
        ],
        out_specs=[
            pl.BlockSpec((_BLK, A), lambda i: (i, 0)),
            pl.BlockSpec((8, A), lambda i: (0, 0)),
            pl.BlockSpec((8, A), lambda i: (0, 0)),
        ],
        out_shape=[
            jax.ShapeDtypeStruct((N, A), jnp.float32),
            jax.ShapeDtypeStruct((8, A), jnp.float32),
            jax.ShapeDtypeStruct((8, A), jnp.float32),
        ],
    )(gathered3, atom_fea, nbr_flat, ws, wne, b8)

    g8 = jnp.broadcast_to(gamma.reshape(1, A), (8, A))
    bt8 = jnp.broadcast_to(beta.reshape(1, A), (8, A))
    out = pl.pallas_call(
        _pass2_body,
        grid=(N // _BLK2,),
        in_specs=[
            pl.BlockSpec((_BLK2, A), lambda i: (i, 0)),
            pl.BlockSpec((8, A), lambda i: (0, 0)),
            pl.BlockSpec((8, A), lambda i: (0, 0)),
            pl.BlockSpec((8, A), lambda i: (0, 0)),
            pl.BlockSpec((8, A), lambda i: (0, 0)),
        ],
        out_specs=pl.BlockSpec((_BLK2, A), lambda i: (i, 0)),
        out_shape=jax.ShapeDtypeStruct((N, A), jnp.float32),
    )(pre, ssum, ssq, g8, bt8)
    return out
